# Initial kernel scaffold; baseline (speedup 1.0000x reference)
import jax
import jax.numpy as jnp
from jax.experimental import pallas as pl


def kernel(h, he, edge_index, batch_num_nodes, params):
    raise NotImplementedError("write your pallas kernel here")


# speedup vs baseline: 1.9275x; 1.9275x over previous
import functools

import jax
import jax.numpy as jnp
import numpy as np
from jax import lax
from jax.experimental import pallas as pl
from jax.experimental.pallas import tpu as pltpu
from jax.experimental.pallas import tpu_sc as plsc

N = 10000
E = 160000
B = 16
MAXLEN = 625
D = 256
DE = 16
MR = 4
H = 4
NUM_LAYERS = 4
DELTA = 0.1

_NC = 2
_NS = 16
_NW = _NC * _NS


def _ln(x, g=None, b=None):
    m = jnp.mean(x, axis=-1, keepdims=True)
    v = jnp.mean((x - m) ** 2, axis=-1, keepdims=True)
    y = (x - m) * lax.rsqrt(v + 1e-5)
    if g is not None:
        y = y * g + b
    return y


def _gather_body(table_hbm, idx_hbm, out_hbm, idx_v, rows_v, sem, *,
                 per_w, chunk, n_ch):
    wid = lax.axis_index("s") * _NC + lax.axis_index("c")
    base = wid * per_w

    def body(j, carry):
        off = base + j * chunk
        pltpu.sync_copy(idx_hbm.at[pl.ds(off, chunk)], idx_v)
        pltpu.async_copy(table_hbm.at[idx_v], rows_v, sem).wait()
        pltpu.sync_copy(rows_v, out_hbm.at[pl.ds(off, chunk)])
        return carry

    lax.fori_loop(0, n_ch, body, 0)


def _sc_gather(table, idx, chunk):
    rows, dt = table.shape
    e = idx.shape[0]
    per_w = e // _NW
    assert per_w * _NW == e and per_w % chunk == 0 and chunk % 8 == 0
    n_ch = per_w // chunk
    mesh = plsc.VectorSubcoreMesh(core_axis_name="c", subcore_axis_name="s")
    f = pl.kernel(
        functools.partial(_gather_body, per_w=per_w, chunk=chunk, n_ch=n_ch),
        mesh=mesh,
        out_type=jax.ShapeDtypeStruct((e, dt), jnp.float32),
        scratch_types=[
            pltpu.VMEM((chunk,), jnp.int32),
            pltpu.VMEM((chunk, dt), jnp.float32),
            pltpu.SemaphoreType.DMA,
        ],
    )
    return f(table, idx)


def _scatter_body(msg_hbm, dst_hbm, zeros_hbm, out_hbm, idx_v, buf_v, acc,
                  *, ec, n_ch, per_tile, rows_per_tile):
    cid = lax.axis_index("c")
    sid = lax.axis_index("s")
    nbase = sid * rows_per_tile
    pltpu.sync_copy(zeros_hbm, acc.at[pl.ds(nbase, rows_per_tile)])
    plsc.subcore_barrier()
    ebase = sid * per_tile

    def body(j, carry):
        off = ebase + j * ec
        pltpu.sync_copy(dst_hbm.at[pl.ds(off, ec)], idx_v)
        pltpu.sync_copy(msg_hbm.at[pl.ds(off, ec), cid], buf_v)
        pltpu.sync_copy(buf_v, acc.at[idx_v], add=True)
        return carry

    lax.fori_loop(0, n_ch, body, 0)
    plsc.subcore_barrier()
    pltpu.sync_copy(acc.at[pl.ds(nbase, rows_per_tile)],
                    out_hbm.at[pl.ds(nbase, rows_per_tile), cid])


def _sc_scatter_add(msg, dst, ec=200):
    e = msg.shape[0]
    msg3 = msg.reshape(e, 2, 128)
    per_tile = e // _NS
    rows_per_tile = N // _NS
    assert per_tile % ec == 0 and ec % 8 == 0
    n_ch = per_tile // ec
    zeros = jnp.zeros((rows_per_tile, 128), jnp.float32)
    mesh = plsc.VectorSubcoreMesh(core_axis_name="c", subcore_axis_name="s")
    f = pl.kernel(
        functools.partial(_scatter_body, ec=ec, n_ch=n_ch, per_tile=per_tile,
                          rows_per_tile=rows_per_tile),
        mesh=mesh,
        out_type=jax.ShapeDtypeStruct((N, 2, 128), jnp.float32),
        scratch_types=[
            pltpu.VMEM((ec,), jnp.int32),
            pltpu.VMEM((ec, 128), jnp.float32),
            pltpu.VMEM_SHARED((N, 128), jnp.float32),
        ],
    )
    return f(msg3, dst, zeros).reshape(N, D)


_EB = 2000


def _edge_msg_body(hs_ref, bd_ref, he_ref, wa, wc, we2, wm1h, wm1e, wm2,
                   he_out, msg_out):
    hs = hs_ref[...]
    he = he_ref[...]
    z = (jnp.dot(hs, wa[...], preferred_element_type=jnp.float32)
         + bd_ref[...][:, :MR * DE]
         + jnp.dot(he, wc[...], preferred_element_type=jnp.float32))
    m = jnp.dot(jax.nn.relu(z), we2[...], preferred_element_type=jnp.float32)
    he2 = _ln(he + m)
    he_out[...] = he2
    p = jax.nn.relu(
        jnp.dot(hs, wm1h[...], preferred_element_type=jnp.float32)
        + jnp.dot(he2, wm1e[...], preferred_element_type=jnp.float32))
    msg_out[...] = jnp.dot(p, wm2[...], preferred_element_type=jnp.float32)


def _tc_edge_msg(hs, bd, he, wa, wc, we2, wm1h, wm1e, wm2):
    grid = (E // _EB,)
    full = lambda *s: pl.BlockSpec(s, lambda i: (0,) * len(s))
    row = lambda *s: pl.BlockSpec(s, lambda i: (i,) + (0,) * (len(s) - 1))
    return pl.pallas_call(
        _edge_msg_body,
        grid=grid,
        in_specs=[
            row(_EB, D), row(_EB, 128), row(_EB, DE),
            full(D, MR * DE), full(DE, MR * DE), full(MR * DE, DE),
            full(D, MR * D), full(DE, MR * D), full(MR * D, D),
        ],
        out_specs=[row(_EB, DE), row(_EB, D)],
        out_shape=[
            jax.ShapeDtypeStruct((E, DE), jnp.float32),
            jax.ShapeDtypeStruct((E, D), jnp.float32),
        ],
    )(hs, bd, he, wa, wc, we2, wm1h, wm1e, wm2)


_NB = 2000


def _node_body(h_ref, agg_ref, wu1h, wu1a, wu2, wbn, h_out, btab_out):
    h = h_ref[...]
    u0 = jax.nn.relu(
        jnp.dot(h, wu1h[...], preferred_element_type=jnp.float32)
        + jnp.dot(agg_ref[...], wu1a[...], preferred_element_type=jnp.float32))
    u = jnp.dot(u0, wu2[...], preferred_element_type=jnp.float32)
    h2 = _ln(h + DELTA * u)
    h_out[...] = h2
    btab_out[...] = jnp.dot(h2, wbn[...], preferred_element_type=jnp.float32)


def _tc_node_update(h, agg, wu1h, wu1a, wu2, wbn):
    grid = (N // _NB,)
    full = lambda *s: pl.BlockSpec(s, lambda i: (0,) * len(s))
    row = lambda *s: pl.BlockSpec(s, lambda i: (i,) + (0,) * (len(s) - 1))
    return pl.pallas_call(
        _node_body,
        grid=grid,
        in_specs=[
            row(_NB, D), row(_NB, D),
            full(D, MR * D), full(D, MR * D), full(MR * D, D),
            full(D, 128),
        ],
        out_specs=[row(_NB, D), row(_NB, 128)],
        out_shape=[
            jax.ShapeDtypeStruct((N, D), jnp.float32),
            jax.ShapeDtypeStruct((N, 128), jnp.float32),
        ],
    )(h, agg, wu1h, wu1a, wu2, wbn)


def _btab_body(h_ref, wb, out_ref):
    out_ref[...] = jnp.dot(h_ref[...], wb[...],
                           preferred_element_type=jnp.float32)


def _tc_btab(h, wb):
    return pl.pallas_call(
        _btab_body,
        grid=(N // _NB,),
        in_specs=[pl.BlockSpec((_NB, D), lambda i: (i, 0)),
                  pl.BlockSpec((D, 128), lambda i: (0, 0))],
        out_specs=pl.BlockSpec((_NB, 128), lambda i: (i, 0)),
        out_shape=jax.ShapeDtypeStruct((N, 128), jnp.float32),
    )(h, wb)


def _tf_body(bnn_ref, x_ref, pn_g, pn_b, wq, bq, wk, bk, wv, bv, wo, bo,
             wf1, bf1, wf2, bf2, l1g, l1b, l2g, l2b, wl, bl, lng, lnb,
             lf_ref, g_ref):
    bidx = pl.program_id(0)
    nb = bnn_ref[bidx]
    x0 = _ln(x_ref[0], pn_g[...], pn_b[...])
    q = jnp.dot(x0, wq[...], preferred_element_type=jnp.float32) + bq[...]
    k = jnp.dot(x0, wk[...], preferred_element_type=jnp.float32) + bk[...]
    v = jnp.dot(x0, wv[...], preferred_element_type=jnp.float32) + bv[...]
    colmask = lax.broadcasted_iota(jnp.int32, (MAXLEN, MAXLEN), 1) >= nb
    dh = D // H
    outs = []
    for hh in range(H):
        sl = slice(hh * dh, (hh + 1) * dh)
        s = lax.dot_general(q[:, sl], k[:, sl], (((1,), (1,)), ((), ())),
                            preferred_element_type=jnp.float32)
        s = s * np.float32(1.0 / np.sqrt(dh))
        s = jnp.where(colmask, np.float32(-1e9), s)
        s = s - jnp.max(s, axis=-1, keepdims=True)
        es = jnp.exp(s)
        a = es / jnp.sum(es, axis=-1, keepdims=True)
        outs.append(jnp.dot(a, v[:, sl], preferred_element_type=jnp.float32))
    o = jnp.concatenate(outs, axis=1)
    o = jnp.dot(o, wo[...], preferred_element_type=jnp.float32) + bo[...]
    x1 = _ln(x0 + o, l1g[...], l1b[...])
    f0 = jax.nn.relu(
        jnp.dot(x1, wf1[...], preferred_element_type=jnp.float32) + bf1[...])
    f = jnp.dot(f0, wf2[...], preferred_element_type=jnp.float32) + bf2[...]
    lf = _ln(x1 + f, l2g[...], l2b[...])
    lf_ref[0] = lf
    rowmask = lax.broadcasted_iota(jnp.int32, (MAXLEN, 1), 0) < nb
    pooled = (jnp.sum(jnp.where(rowmask, lf, 0.0), axis=0, keepdims=True)
              / nb.astype(jnp.float32))
    g = _ln(jnp.dot(pooled, wl[...], preferred_element_type=jnp.float32)
            + bl[...], lng[...], lnb[...])
    g_ref[0] = g


def _tc_transformer(h, bnn, tf, wl, bl, lng, lnb, pn_g, pn_b):
    xb = h.reshape(B, MAXLEN, D)
    r2 = lambda a: a.reshape(1, -1)
    full = lambda *s: pl.BlockSpec(s, lambda i: (0,) * len(s))
    args = [
        xb, r2(pn_g), r2(pn_b),
        tf["Wq"], r2(tf["bq"]), tf["Wk"], r2(tf["bk"]),
        tf["Wv"], r2(tf["bv"]), tf["Wo"], r2(tf["bo"]),
        tf["Wf1"], r2(tf["bf1"]), tf["Wf2"], r2(tf["bf2"]),
        r2(tf["ln1_g"]), r2(tf["ln1_b"]), r2(tf["ln2_g"]), r2(tf["ln2_b"]),
        wl, r2(bl), r2(lng), r2(lnb),
    ]
    in_specs = [pl.BlockSpec(memory_space=pltpu.SMEM),
                pl.BlockSpec((1, MAXLEN, D), lambda i: (i, 0, 0))]
    in_specs += [full(*a.shape) for a in args[1:]]
    lf, g = pl.pallas_call(
        _tf_body,
        grid=(B,),
        in_specs=in_specs,
        out_specs=[pl.BlockSpec((1, MAXLEN, D), lambda i: (i, 0, 0)),
                   pl.BlockSpec((1, 1, D), lambda i: (i, 0, 0))],
        out_shape=[
            jax.ShapeDtypeStruct((B, MAXLEN, D), jnp.float32),
            jax.ShapeDtypeStruct((B, 1, D), jnp.float32),
        ],
    )(bnn, *args)
    return lf.reshape(N, D), g.reshape(B, D)


def kernel(h, he, edge_index, batch_num_nodes, params):
    src = edge_index[0]
    dst = edge_index[1]

    pad_b = lambda w: jnp.pad(w, ((0, 0), (0, 128 - MR * DE)))
    wb_next = pad_b(params["edge1"]["We1"][D:2 * D])
    btab = _tc_btab(h, pad_b(params["edge0"]["We1"][D:2 * D]))

    for l in range(NUM_LAYERS):
        pe = params["edge0" if l == 0 else "edge1"]
        pn = params["node0" if l == 0 else "node1"]
        wa = pe["We1"][:D]
        wc = pe["We1"][2 * D:]
        wm1h = pn["Wm1"][:D]
        wm1e = pn["Wm1"][D:]
        wu1h = pn["Wu1"][:D]
        wu1a = pn["Wu1"][D:]

        hs = _sc_gather(h, src, chunk=200)
        bd = _sc_gather(btab, dst, chunk=200)
        he, msg = _tc_edge_msg(hs, bd, he, wa, wc, we2=pe["We2"],
                               wm1h=wm1h, wm1e=wm1e, wm2=pn["Wm2"])
        agg = _sc_scatter_add(msg, dst)
        h, btab = _tc_node_update(h, agg, wu1h, wu1a, pn["Wu2"], wb_next)

    local_feat, global_feat = _tc_transformer(
        h, batch_num_nodes, params["tf"], params["Wl"], params["bl"],
        params["ln_g"], params["ln_b"], params["pn_g"], params["pn_b"])
    return local_feat, global_feat


# speedup vs baseline: 2.0855x; 1.0820x over previous
import functools

import jax
import jax.numpy as jnp
import numpy as np
from jax import lax
from jax.experimental import pallas as pl
from jax.experimental.pallas import tpu as pltpu
from jax.experimental.pallas import tpu_sc as plsc

N = 10000
E = 160000
B = 16
MAXLEN = 625
D = 256
DE = 16
MR = 4
H = 4
NUM_LAYERS = 4
DELTA = 0.1

_NC = 2
_NS = 16
_NW = _NC * _NS


def _bdot(a, b):
    return jnp.dot(a.astype(jnp.bfloat16), b.astype(jnp.bfloat16),
                   preferred_element_type=jnp.float32)


def _ln(x, g=None, b=None):
    m = jnp.mean(x, axis=-1, keepdims=True)
    v = jnp.mean((x - m) ** 2, axis=-1, keepdims=True)
    y = (x - m) * lax.rsqrt(v + 1e-5)
    if g is not None:
        y = y * g + b
    return y


def _gather_body(table_hbm, idx_hbm, out_hbm, idx_v, rows_v, sem, *,
                 per_w, chunk, n_ch):
    wid = lax.axis_index("s") * _NC + lax.axis_index("c")
    base = wid * per_w

    def body(j, carry):
        off = base + j * chunk
        pltpu.sync_copy(idx_hbm.at[pl.ds(off, chunk)], idx_v)
        pltpu.async_copy(table_hbm.at[idx_v], rows_v, sem).wait()
        pltpu.sync_copy(rows_v, out_hbm.at[pl.ds(off, chunk)])
        return carry

    lax.fori_loop(0, n_ch, body, 0)


def _sc_gather(table, idx, chunk):
    rows, dt = table.shape
    e = idx.shape[0]
    per_w = e // _NW
    assert per_w * _NW == e and per_w % chunk == 0 and chunk % 8 == 0
    n_ch = per_w // chunk
    mesh = plsc.VectorSubcoreMesh(core_axis_name="c", subcore_axis_name="s")
    f = pl.kernel(
        functools.partial(_gather_body, per_w=per_w, chunk=chunk, n_ch=n_ch),
        mesh=mesh,
        out_type=jax.ShapeDtypeStruct((e, dt), jnp.float32),
        scratch_types=[
            pltpu.VMEM((chunk,), jnp.int32),
            pltpu.VMEM((chunk, dt), jnp.float32),
            pltpu.SemaphoreType.DMA,
        ],
    )
    return f(table, idx)


def _scatter_body(msg_hbm, dst_hbm, zeros_hbm, out_hbm, idx_v, buf_v, acc,
                  *, ec, n_ch, per_tile, rows_per_tile):
    cid = lax.axis_index("c")
    sid = lax.axis_index("s")
    nbase = pl.multiple_of(sid * 624, 8)
    pltpu.sync_copy(zeros_hbm, acc.at[pl.ds(nbase, 640)])
    plsc.subcore_barrier()
    ebase = sid * per_tile

    def body(j, carry):
        off = pl.multiple_of(ebase + j * ec, 8)
        pltpu.sync_copy(dst_hbm.at[pl.ds(off, ec)], idx_v)
        pltpu.sync_copy(msg_hbm.at[cid, pl.ds(off, ec)], buf_v)
        pltpu.sync_copy(buf_v, acc.at[idx_v], add=True)
        return carry

    lax.fori_loop(0, n_ch, body, 0)
    plsc.subcore_barrier()
    pltpu.sync_copy(acc.at[pl.ds(nbase, 640)],
                    out_hbm.at[cid, pl.ds(nbase, 640)])


def _sc_scatter_add(msg3, dst, ec=200):
    e = msg3.shape[1]
    per_tile = e // _NS
    rows_per_tile = N // _NS
    assert per_tile % ec == 0 and ec % 8 == 0
    assert 624 * (_NS - 1) + 640 == N
    n_ch = per_tile // ec
    zeros = jnp.zeros((640, 128), jnp.float32)
    mesh = plsc.VectorSubcoreMesh(core_axis_name="c", subcore_axis_name="s")
    f = pl.kernel(
        functools.partial(_scatter_body, ec=ec, n_ch=n_ch, per_tile=per_tile,
                          rows_per_tile=rows_per_tile),
        mesh=mesh,
        out_type=jax.ShapeDtypeStruct((2, N, 128), jnp.float32),
        scratch_types=[
            pltpu.VMEM((ec,), jnp.int32),
            pltpu.VMEM((ec, 128), jnp.float32),
            pltpu.VMEM_SHARED((N, 128), jnp.float32),
        ],
    )
    return f(msg3, dst, zeros)


_EB = 2000


def _edge_msg_body(hs_ref, bd_ref, he_ref, wa, wc, we2, wm1h, wm1e, wm2,
                   he_out, msg_out):
    hs = hs_ref[...]
    he = he_ref[...]
    z = (_bdot(hs, wa[...])
         + bd_ref[...][:, :MR * DE]
         + _bdot(he, wc[...]))
    m = _bdot(jax.nn.relu(z), we2[...])
    he2 = _ln(he + m)
    he_out[...] = he2
    p = jax.nn.relu(_bdot(hs, wm1h[...]) + _bdot(he2, wm1e[...]))
    msgv = _bdot(p, wm2[...])
    msg_out[0] = msgv[:, :128]
    msg_out[1] = msgv[:, 128:]


def _tc_edge_msg(hs, bd, he, wa, wc, we2, wm1h, wm1e, wm2):
    grid = (E // _EB,)
    full = lambda *s: pl.BlockSpec(s, lambda i: (0,) * len(s))
    row = lambda *s: pl.BlockSpec(s, lambda i: (i,) + (0,) * (len(s) - 1))
    return pl.pallas_call(
        _edge_msg_body,
        grid=grid,
        in_specs=[
            row(_EB, D), row(_EB, 128), row(_EB, DE),
            full(D, MR * DE), full(DE, MR * DE), full(MR * DE, DE),
            full(D, MR * D), full(DE, MR * D), full(MR * D, D),
        ],
        out_specs=[row(_EB, DE),
                   pl.BlockSpec((2, _EB, 128), lambda i: (0, i, 0))],
        out_shape=[
            jax.ShapeDtypeStruct((E, DE), jnp.float32),
            jax.ShapeDtypeStruct((2, E, 128), jnp.float32),
        ],
    )(hs, bd, he, wa, wc, we2, wm1h, wm1e, wm2)


_NB = 2000


def _node_body(h_ref, agg_ref, wu1h, wu1a, wu2, wbn, h_out, btab_out):
    h = h_ref[...]
    u0 = jax.nn.relu(_bdot(h, wu1h[...])
                     + _bdot(agg_ref[0], wu1a[...][:128])
                     + _bdot(agg_ref[1], wu1a[...][128:]))
    u = _bdot(u0, wu2[...])
    h2 = _ln(h + DELTA * u)
    h_out[...] = h2
    btab_out[...] = _bdot(h2, wbn[...])


def _tc_node_update(h, agg, wu1h, wu1a, wu2, wbn):
    grid = (N // _NB,)
    full = lambda *s: pl.BlockSpec(s, lambda i: (0,) * len(s))
    row = lambda *s: pl.BlockSpec(s, lambda i: (i,) + (0,) * (len(s) - 1))
    return pl.pallas_call(
        _node_body,
        grid=grid,
        in_specs=[
            row(_NB, D),
            pl.BlockSpec((2, _NB, 128), lambda i: (0, i, 0)),
            full(D, MR * D), full(D, MR * D), full(MR * D, D),
            full(D, 128),
        ],
        out_specs=[row(_NB, D), row(_NB, 128)],
        out_shape=[
            jax.ShapeDtypeStruct((N, D), jnp.float32),
            jax.ShapeDtypeStruct((N, 128), jnp.float32),
        ],
    )(h, agg, wu1h, wu1a, wu2, wbn)


def _btab_body(h_ref, wb, out_ref):
    out_ref[...] = jnp.dot(h_ref[...], wb[...],
                           preferred_element_type=jnp.float32)


def _tc_btab(h, wb):
    return pl.pallas_call(
        _btab_body,
        grid=(N // _NB,),
        in_specs=[pl.BlockSpec((_NB, D), lambda i: (i, 0)),
                  pl.BlockSpec((D, 128), lambda i: (0, 0))],
        out_specs=pl.BlockSpec((_NB, 128), lambda i: (i, 0)),
        out_shape=jax.ShapeDtypeStruct((N, 128), jnp.float32),
    )(h, wb)


def _tf_body(bnn_ref, x_ref, pn_g, pn_b, wq, bq, wk, bk, wv, bv, wo, bo,
             wf1, bf1, wf2, bf2, l1g, l1b, l2g, l2b, wl, bl, lng, lnb,
             lf_ref, g_ref):
    bidx = pl.program_id(0)
    nb = bnn_ref[bidx]
    x0 = _ln(x_ref[0], pn_g[...], pn_b[...])
    q = _bdot(x0, wq[...]) + bq[...]
    k = _bdot(x0, wk[...]) + bk[...]
    v = _bdot(x0, wv[...]) + bv[...]
    colmask = lax.broadcasted_iota(jnp.int32, (MAXLEN, MAXLEN), 1) >= nb
    dh = D // H
    outs = []
    for hh in range(H):
        sl = slice(hh * dh, (hh + 1) * dh)
        s = lax.dot_general(q[:, sl].astype(jnp.bfloat16),
                            k[:, sl].astype(jnp.bfloat16),
                            (((1,), (1,)), ((), ())),
                            preferred_element_type=jnp.float32)
        s = s * np.float32(1.0 / np.sqrt(dh))
        s = jnp.where(colmask, np.float32(-1e9), s)
        s = s - jnp.max(s, axis=-1, keepdims=True)
        es = jnp.exp(s)
        a = es / jnp.sum(es, axis=-1, keepdims=True)
        outs.append(_bdot(a, v[:, sl]))
    o = jnp.concatenate(outs, axis=1)
    o = _bdot(o, wo[...]) + bo[...]
    x1 = _ln(x0 + o, l1g[...], l1b[...])
    f0 = jax.nn.relu(_bdot(x1, wf1[...]) + bf1[...])
    f = _bdot(f0, wf2[...]) + bf2[...]
    lf = _ln(x1 + f, l2g[...], l2b[...])
    lf_ref[0] = lf
    rowmask = lax.broadcasted_iota(jnp.int32, (MAXLEN, 1), 0) < nb
    pooled = (jnp.sum(jnp.where(rowmask, lf, 0.0), axis=0, keepdims=True)
              / nb.astype(jnp.float32))
    g = _ln(_bdot(pooled, wl[...]) + bl[...], lng[...], lnb[...])
    g_ref[0] = g


def _tc_transformer(h, bnn, tf, wl, bl, lng, lnb, pn_g, pn_b):
    xb = h.reshape(B, MAXLEN, D)
    r2 = lambda a: a.reshape(1, -1)
    full = lambda *s: pl.BlockSpec(s, lambda i: (0,) * len(s))
    args = [
        xb, r2(pn_g), r2(pn_b),
        tf["Wq"], r2(tf["bq"]), tf["Wk"], r2(tf["bk"]),
        tf["Wv"], r2(tf["bv"]), tf["Wo"], r2(tf["bo"]),
        tf["Wf1"], r2(tf["bf1"]), tf["Wf2"], r2(tf["bf2"]),
        r2(tf["ln1_g"]), r2(tf["ln1_b"]), r2(tf["ln2_g"]), r2(tf["ln2_b"]),
        wl, r2(bl), r2(lng), r2(lnb),
    ]
    in_specs = [pl.BlockSpec(memory_space=pltpu.SMEM),
                pl.BlockSpec((1, MAXLEN, D), lambda i: (i, 0, 0))]
    in_specs += [full(*a.shape) for a in args[1:]]
    lf, g = pl.pallas_call(
        _tf_body,
        grid=(B,),
        in_specs=in_specs,
        out_specs=[pl.BlockSpec((1, MAXLEN, D), lambda i: (i, 0, 0)),
                   pl.BlockSpec((1, 1, D), lambda i: (i, 0, 0))],
        out_shape=[
            jax.ShapeDtypeStruct((B, MAXLEN, D), jnp.float32),
            jax.ShapeDtypeStruct((B, 1, D), jnp.float32),
        ],
    )(bnn, *args)
    return lf.reshape(N, D), g.reshape(B, D)


def kernel(h, he, edge_index, batch_num_nodes, params):
    src = edge_index[0]
    dst = edge_index[1]

    pad_b = lambda w: jnp.pad(w, ((0, 0), (0, 128 - MR * DE)))
    wb_next = pad_b(params["edge1"]["We1"][D:2 * D])
    btab = _tc_btab(h, pad_b(params["edge0"]["We1"][D:2 * D]))

    for l in range(NUM_LAYERS):
        pe = params["edge0" if l == 0 else "edge1"]
        pn = params["node0" if l == 0 else "node1"]
        wa = pe["We1"][:D]
        wc = pe["We1"][2 * D:]
        wm1h = pn["Wm1"][:D]
        wm1e = pn["Wm1"][D:]
        wu1h = pn["Wu1"][:D]
        wu1a = pn["Wu1"][D:]

        hs = _sc_gather(h, src, chunk=200)
        bd = _sc_gather(btab, dst, chunk=200)
        he, msg = _tc_edge_msg(hs, bd, he, wa, wc, we2=pe["We2"],
                               wm1h=wm1h, wm1e=wm1e, wm2=pn["Wm2"])
        agg = _sc_scatter_add(msg, dst)
        h, btab = _tc_node_update(h, agg, wu1h, wu1a, pn["Wu2"], wb_next)

    local_feat, global_feat = _tc_transformer(
        h, batch_num_nodes, params["tf"], params["Wl"], params["bl"],
        params["ln_g"], params["ln_b"], params["pn_g"], params["pn_b"])
    return local_feat, global_feat


# speedup vs baseline: 2.3932x; 1.1476x over previous
import functools

import jax
import jax.numpy as jnp
import numpy as np
from jax import lax
from jax.experimental import pallas as pl
from jax.experimental.pallas import tpu as pltpu
from jax.experimental.pallas import tpu_sc as plsc

N = 10000
E = 160000
B = 16
MAXLEN = 625
D = 256
DE = 16
MR = 4
H = 4
NUM_LAYERS = 4
DELTA = 0.1

_NC = 2
_NS = 16
_NW = _NC * _NS


def _bdot(a, b, out=jnp.float32):
    return jnp.dot(a.astype(jnp.bfloat16), b.astype(jnp.bfloat16),
                   preferred_element_type=out)


def _ln(x, g=None, b=None):
    m = jnp.mean(x, axis=-1, keepdims=True)
    v = jnp.mean((x - m) ** 2, axis=-1, keepdims=True)
    y = (x - m) * lax.rsqrt(v + 1e-5)
    if g is not None:
        y = y * g + b
    return y


def _gather_body(table_hbm, idx_hbm, out_hbm, idx_v, rows_v, sem, *,
                 per_w, chunk, n_ch):
    wid = lax.axis_index("s") * _NC + lax.axis_index("c")
    base = wid * per_w

    def body(j, carry):
        off = base + j * chunk
        pltpu.sync_copy(idx_hbm.at[pl.ds(off, chunk)], idx_v)
        pltpu.async_copy(table_hbm.at[idx_v], rows_v, sem).wait()
        pltpu.sync_copy(rows_v, out_hbm.at[pl.ds(off, chunk)])
        return carry

    lax.fori_loop(0, n_ch, body, 0)


def _sc_gather(table, idx, chunk):
    rows, dt = table.shape
    e = idx.shape[0]
    per_w = e // _NW
    assert per_w * _NW == e and per_w % chunk == 0 and chunk % 8 == 0
    n_ch = per_w // chunk
    mesh = plsc.VectorSubcoreMesh(core_axis_name="c", subcore_axis_name="s")
    f = pl.kernel(
        functools.partial(_gather_body, per_w=per_w, chunk=chunk, n_ch=n_ch),
        mesh=mesh,
        out_type=jax.ShapeDtypeStruct((e, dt), jnp.float32),
        scratch_types=[
            pltpu.VMEM((chunk,), jnp.int32),
            pltpu.VMEM((chunk, dt), jnp.float32),
            pltpu.SemaphoreType.DMA,
        ],
    )
    return f(table, idx)


def _scatter_body(msg_hbm, dst_hbm, zeros_hbm, out_hbm, idx_v, buf_v, acc,
                  *, ec, n_ch, per_tile, rows_per_tile):
    cid = lax.axis_index("c")
    sid = lax.axis_index("s")
    nbase = pl.multiple_of(sid * 624, 8)
    pltpu.sync_copy(zeros_hbm, acc.at[pl.ds(nbase, 640)])
    plsc.subcore_barrier()
    ebase = sid * per_tile

    def body(j, carry):
        off = pl.multiple_of(ebase + j * ec, 8)
        pltpu.sync_copy(dst_hbm.at[pl.ds(off, ec)], idx_v)
        pltpu.sync_copy(msg_hbm.at[cid, pl.ds(off, ec)], buf_v)
        pltpu.sync_copy(buf_v, acc.at[idx_v], add=True)
        return carry

    lax.fori_loop(0, n_ch, body, 0)
    plsc.subcore_barrier()
    pltpu.sync_copy(acc.at[pl.ds(nbase, 640)],
                    out_hbm.at[cid, pl.ds(nbase, 640)])


def _sc_scatter_add(msg3, dst, ec=200):
    e = msg3.shape[1]
    per_tile = e // _NS
    rows_per_tile = N // _NS
    assert per_tile % ec == 0 and ec % 8 == 0
    assert 624 * (_NS - 1) + 640 == N
    n_ch = per_tile // ec
    zeros = jnp.zeros((640, 128), jnp.float32)
    mesh = plsc.VectorSubcoreMesh(core_axis_name="c", subcore_axis_name="s")
    f = pl.kernel(
        functools.partial(_scatter_body, ec=ec, n_ch=n_ch, per_tile=per_tile,
                          rows_per_tile=rows_per_tile),
        mesh=mesh,
        out_type=jax.ShapeDtypeStruct((2, N, 128), jnp.float32),
        scratch_types=[
            pltpu.VMEM((ec,), jnp.int32),
            pltpu.VMEM((ec, 128), jnp.float32),
            pltpu.VMEM_SHARED((N, 128), jnp.float32),
        ],
    )
    return f(msg3, dst, zeros)


_EB = 2000


def _edge_msg_body(hs_ref, bd_ref, he_ref, wa, wc, we2, wm1h, wm1e, wm2,
                   he_out, msg_out):
    hs = hs_ref[...]
    he = he_ref[...]
    f32 = jnp.float32
    z = (jnp.dot(hs, wa[...], preferred_element_type=f32)
         + bd_ref[...][:, :MR * DE]
         + jnp.dot(he, wc[...], preferred_element_type=f32))
    m = jnp.dot(jax.nn.relu(z), we2[...], preferred_element_type=f32)
    he2 = _ln(he + m)
    he_out[...] = he2
    p = jax.nn.relu(
        jnp.dot(hs, wm1h[...], preferred_element_type=f32)
        + jnp.dot(he2, wm1e[...], preferred_element_type=f32))
    msgv = jnp.dot(p, wm2[...], preferred_element_type=f32)
    msg_out[0] = msgv[:, :128]
    msg_out[1] = msgv[:, 128:]


def _tc_edge_msg(hs, bd, he, wa, wc, we2, wm1h, wm1e, wm2):
    grid = (E // _EB,)
    full = lambda *s: pl.BlockSpec(s, lambda i: (0,) * len(s))
    row = lambda *s: pl.BlockSpec(s, lambda i: (i,) + (0,) * (len(s) - 1))
    return pl.pallas_call(
        _edge_msg_body,
        grid=grid,
        in_specs=[
            row(_EB, D), row(_EB, 128), row(_EB, DE),
            full(D, MR * DE), full(DE, MR * DE), full(MR * DE, DE),
            full(D, MR * D), full(DE, MR * D), full(MR * D, D),
        ],
        out_specs=[row(_EB, DE),
                   pl.BlockSpec((2, _EB, 128), lambda i: (0, i, 0))],
        out_shape=[
            jax.ShapeDtypeStruct((E, DE), jnp.float32),
            jax.ShapeDtypeStruct((2, E, 128), jnp.float32),
        ],
    )(hs, bd, he, wa, wc, we2, wm1h, wm1e, wm2)


_NB = 2000


def _node_body(h_ref, agg_ref, wu1h, wu1a, wu2, wbn, h_out, btab_out):
    h = h_ref[...]
    f32 = jnp.float32
    u0 = jax.nn.relu(jnp.dot(h, wu1h[...], preferred_element_type=f32)
                     + jnp.dot(agg_ref[0], wu1a[...][:128],
                               preferred_element_type=f32)
                     + jnp.dot(agg_ref[1], wu1a[...][128:],
                               preferred_element_type=f32))
    u = jnp.dot(u0, wu2[...], preferred_element_type=f32)
    h2 = _ln(h + DELTA * u)
    h_out[...] = h2
    btab_out[...] = jnp.dot(h2, wbn[...], preferred_element_type=jnp.float32)


def _tc_node_update(h, agg, wu1h, wu1a, wu2, wbn):
    grid = (N // _NB,)
    full = lambda *s: pl.BlockSpec(s, lambda i: (0,) * len(s))
    row = lambda *s: pl.BlockSpec(s, lambda i: (i,) + (0,) * (len(s) - 1))
    return pl.pallas_call(
        _node_body,
        grid=grid,
        in_specs=[
            row(_NB, D),
            pl.BlockSpec((2, _NB, 128), lambda i: (0, i, 0)),
            full(D, MR * D), full(D, MR * D), full(MR * D, D),
            full(D, 128),
        ],
        out_specs=[row(_NB, D), row(_NB, 128)],
        out_shape=[
            jax.ShapeDtypeStruct((N, D), jnp.float32),
            jax.ShapeDtypeStruct((N, 128), jnp.float32),
        ],
    )(h, agg, wu1h, wu1a, wu2, wbn)


def _btab_body(h_ref, wb, out_ref):
    out_ref[...] = jnp.dot(h_ref[...], wb[...],
                           preferred_element_type=jnp.float32)


def _tc_btab(h, wb):
    return pl.pallas_call(
        _btab_body,
        grid=(N // _NB,),
        in_specs=[pl.BlockSpec((_NB, D), lambda i: (i, 0)),
                  pl.BlockSpec((D, 128), lambda i: (0, 0))],
        out_specs=pl.BlockSpec((_NB, 128), lambda i: (i, 0)),
        out_shape=jax.ShapeDtypeStruct((N, 128), jnp.float32),
    )(h, wb)


def _tf_body(bnn_ref, x_ref, pn_g, pn_b, wq, bq, wk, bk, wv, bv, wo, bo,
             wf1, bf1, wf2, bf2, l1g, l1b, l2g, l2b, wl, bl, lng, lnb,
             lf_ref, g_ref):
    bidx = pl.program_id(0)
    nb = bnn_ref[bidx]
    x0 = _ln(x_ref[0], pn_g[...], pn_b[...])
    q = _bdot(x0, wq[...]) + bq[...]
    k = _bdot(x0, wk[...]) + bk[...]
    v = _bdot(x0, wv[...]) + bv[...]
    colmask = lax.broadcasted_iota(jnp.int32, (MAXLEN, MAXLEN), 1) >= nb
    dh = D // H
    outs = []
    for hh in range(H):
        sl = slice(hh * dh, (hh + 1) * dh)
        s = lax.dot_general(q[:, sl].astype(jnp.bfloat16),
                            k[:, sl].astype(jnp.bfloat16),
                            (((1,), (1,)), ((), ())),
                            preferred_element_type=jnp.float32)
        s = s * np.float32(1.0 / np.sqrt(dh))
        s = jnp.where(colmask, np.float32(-1e9), s)
        s = s - jnp.max(s, axis=-1, keepdims=True)
        es = jnp.exp(s)
        a = es / jnp.sum(es, axis=-1, keepdims=True)
        outs.append(_bdot(a, v[:, sl]))
    o = jnp.concatenate(outs, axis=1)
    o = _bdot(o, wo[...]) + bo[...]
    x1 = _ln(x0 + o, l1g[...], l1b[...])
    f0 = jax.nn.relu(_bdot(x1, wf1[...]) + bf1[...])
    f = _bdot(f0, wf2[...]) + bf2[...]
    lf = _ln(x1 + f, l2g[...], l2b[...])
    lf_ref[0] = lf
    rowmask = lax.broadcasted_iota(jnp.int32, (MAXLEN, 1), 0) < nb
    pooled = (jnp.sum(jnp.where(rowmask, lf, 0.0), axis=0, keepdims=True)
              / nb.astype(jnp.float32))
    g = _ln(_bdot(pooled, wl[...]) + bl[...], lng[...], lnb[...])
    g_ref[0] = g


def _tc_transformer(h, bnn, tf, wl, bl, lng, lnb, pn_g, pn_b):
    xb = h.reshape(B, MAXLEN, D)
    r2 = lambda a: a.reshape(1, -1)
    full = lambda *s: pl.BlockSpec(s, lambda i: (0,) * len(s))
    args = [
        xb, r2(pn_g), r2(pn_b),
        tf["Wq"], r2(tf["bq"]), tf["Wk"], r2(tf["bk"]),
        tf["Wv"], r2(tf["bv"]), tf["Wo"], r2(tf["bo"]),
        tf["Wf1"], r2(tf["bf1"]), tf["Wf2"], r2(tf["bf2"]),
        r2(tf["ln1_g"]), r2(tf["ln1_b"]), r2(tf["ln2_g"]), r2(tf["ln2_b"]),
        wl, r2(bl), r2(lng), r2(lnb),
    ]
    in_specs = [pl.BlockSpec(memory_space=pltpu.SMEM),
                pl.BlockSpec((1, MAXLEN, D), lambda i: (i, 0, 0))]
    in_specs += [full(*a.shape) for a in args[1:]]
    lf, g = pl.pallas_call(
        _tf_body,
        grid=(B,),
        in_specs=in_specs,
        out_specs=[pl.BlockSpec((1, MAXLEN, D), lambda i: (i, 0, 0)),
                   pl.BlockSpec((1, 1, D), lambda i: (i, 0, 0))],
        out_shape=[
            jax.ShapeDtypeStruct((B, MAXLEN, D), jnp.float32),
            jax.ShapeDtypeStruct((B, 1, D), jnp.float32),
        ],
    )(bnn, *args)
    return lf.reshape(N, D), g.reshape(B, D)


def kernel(h, he, edge_index, batch_num_nodes, params):
    src = edge_index[0]
    dst = edge_index[1]

    pad_b = lambda w: jnp.pad(w, ((0, 0), (0, 128 - MR * DE)))
    wb_next = pad_b(params["edge1"]["We1"][D:2 * D])
    btab = _tc_btab(h, pad_b(params["edge0"]["We1"][D:2 * D]))

    for l in range(NUM_LAYERS):
        pe = params["edge0" if l == 0 else "edge1"]
        pn = params["node0" if l == 0 else "node1"]
        wa = pe["We1"][:D]
        wc = pe["We1"][2 * D:]
        wm1h = pn["Wm1"][:D]
        wm1e = pn["Wm1"][D:]
        wu1h = pn["Wu1"][:D]
        wu1a = pn["Wu1"][D:]

        hs = _sc_gather(h, src, chunk=200)
        bd = _sc_gather(btab, dst, chunk=200)
        he, msg = _tc_edge_msg(hs, bd, he, wa, wc, we2=pe["We2"],
                               wm1h=wm1h, wm1e=wm1e, wm2=pn["Wm2"])
        agg = _sc_scatter_add(msg, dst)
        h, btab = _tc_node_update(h, agg, wu1h, wu1a, pn["Wu2"], wb_next)

    local_feat, global_feat = _tc_transformer(
        h, batch_num_nodes, params["tf"], params["Wl"], params["bl"],
        params["ln_g"], params["ln_b"], params["pn_g"], params["pn_b"])
    return local_feat, global_feat


# speedup vs baseline: 2.9120x; 1.2168x over previous
import functools

import jax
import jax.numpy as jnp
import numpy as np
from jax import lax
from jax.experimental import pallas as pl
from jax.experimental.pallas import tpu as pltpu
from jax.experimental.pallas import tpu_sc as plsc

N = 10000
E = 160000
B = 16
MAXLEN = 625
D = 256
DE = 16
MR = 4
H = 4
NUM_LAYERS = 4
DELTA = 0.1

_NC = 2
_NS = 16
_NW = _NC * _NS


def _bdot(a, b, out=jnp.float32):
    return jnp.dot(a.astype(jnp.bfloat16), b.astype(jnp.bfloat16),
                   preferred_element_type=out)


def _ln(x, g=None, b=None):
    m = jnp.mean(x, axis=-1, keepdims=True)
    v = jnp.mean((x - m) ** 2, axis=-1, keepdims=True)
    y = (x - m) * lax.rsqrt(v + 1e-5)
    if g is not None:
        y = y * g + b
    return y


def _gather_body(table_hbm, idx_hbm, out_hbm, idx_v, rows_v, sem, *,
                 per_w, chunk, n_ch):
    wid = lax.axis_index("s") * _NC + lax.axis_index("c")
    base = wid * per_w

    def body(j, carry):
        off = base + j * chunk
        pltpu.sync_copy(idx_hbm.at[pl.ds(off, chunk)], idx_v)
        pltpu.async_copy(table_hbm.at[idx_v], rows_v, sem).wait()
        pltpu.sync_copy(rows_v, out_hbm.at[pl.ds(off, chunk)])
        return carry

    lax.fori_loop(0, n_ch, body, 0)


def _sc_gather(table, idx, chunk):
    rows, dt = table.shape
    e = idx.shape[0]
    per_w = e // _NW
    assert per_w * _NW == e and per_w % chunk == 0 and chunk % 8 == 0
    n_ch = per_w // chunk
    mesh = plsc.VectorSubcoreMesh(core_axis_name="c", subcore_axis_name="s")
    f = pl.kernel(
        functools.partial(_gather_body, per_w=per_w, chunk=chunk, n_ch=n_ch),
        mesh=mesh,
        out_type=jax.ShapeDtypeStruct((e, dt), jnp.float32),
        scratch_types=[
            pltpu.VMEM((chunk,), jnp.int32),
            pltpu.VMEM((chunk, dt), jnp.float32),
            pltpu.SemaphoreType.DMA,
        ],
    )
    return f(table, idx)


def _scatter_body(msg_hbm, dst_hbm, init_hbm, out_hbm, idx_v, buf_v, acc,
                  *, ec, n_ch, per_tile, rows_per_tile):
    cid = lax.axis_index("c")
    sid = lax.axis_index("s")
    nbase = pl.multiple_of(sid * 624, 8)
    pltpu.sync_copy(init_hbm.at[cid, pl.ds(nbase, 640)], acc.at[pl.ds(nbase, 640)])
    plsc.subcore_barrier()
    ebase = sid * per_tile

    def body(j, carry):
        off = pl.multiple_of(ebase + j * ec, 8)
        pltpu.sync_copy(dst_hbm.at[pl.ds(off, ec)], idx_v)
        pltpu.sync_copy(msg_hbm.at[cid, pl.ds(off, ec)], buf_v)
        pltpu.sync_copy(buf_v, acc.at[idx_v], add=True)
        return carry

    lax.fori_loop(0, n_ch, body, 0)
    plsc.subcore_barrier()
    pltpu.sync_copy(acc.at[pl.ds(nbase, 640)],
                    out_hbm.at[cid, pl.ds(nbase, 640)])


def _sc_scatter_add(msg3, dst, init, ec=200):
    e = msg3.shape[1]
    per_tile = e // _NS
    rows_per_tile = N // _NS
    assert per_tile % ec == 0 and ec % 8 == 0
    assert 624 * (_NS - 1) + 640 == N
    n_ch = per_tile // ec
    mesh = plsc.VectorSubcoreMesh(core_axis_name="c", subcore_axis_name="s")
    f = pl.kernel(
        functools.partial(_scatter_body, ec=ec, n_ch=n_ch, per_tile=per_tile,
                          rows_per_tile=rows_per_tile),
        mesh=mesh,
        out_type=jax.ShapeDtypeStruct((2, N, 128), jnp.float32),
        scratch_types=[
            pltpu.VMEM((ec,), jnp.int32),
            pltpu.VMEM((ec, 128), jnp.float32),
            pltpu.VMEM_SHARED((N, 128), jnp.float32),
        ],
    )
    return f(msg3, dst, init)


_EB = 1600


def _edge_msg_body(hs_ref, bd_ref, he_ref, wa, wc, we2, wm1h, wm1e, wm2,
                   he_out, msg_out):
    hs = hs_ref[...]
    he = he_ref[...]
    f32 = jnp.float32
    z = (jnp.dot(hs, wa[...], preferred_element_type=f32)
         + bd_ref[...][:, :MR * DE]
         + jnp.dot(he, wc[...], preferred_element_type=f32))
    m = jnp.dot(jax.nn.relu(z), we2[...], preferred_element_type=f32)
    he2 = _ln(he + m)
    he_out[...] = he2
    p = jax.nn.relu(
        jnp.dot(hs, wm1h[...], preferred_element_type=f32)
        + jnp.dot(he2, wm1e[...], preferred_element_type=f32))
    msgv = jnp.dot(p, wm2[...], preferred_element_type=f32)
    msg_out[0] = msgv[:, :128]
    msg_out[1] = msgv[:, 128:]


def _tc_edge_msg(hs, bd, he, wa, wc, we2, wm1h, wm1e, wm2):
    e = hs.shape[0]
    grid = (e // _EB,)
    full = lambda *s: pl.BlockSpec(s, lambda i: (0,) * len(s))
    row = lambda *s: pl.BlockSpec(s, lambda i: (i,) + (0,) * (len(s) - 1))
    return pl.pallas_call(
        _edge_msg_body,
        grid=grid,
        in_specs=[
            row(_EB, D), row(_EB, 128), row(_EB, DE),
            full(D, MR * DE), full(DE, MR * DE), full(MR * DE, DE),
            full(D, MR * D), full(DE, MR * D), full(MR * D, D),
        ],
        out_specs=[row(_EB, DE),
                   pl.BlockSpec((2, _EB, 128), lambda i: (0, i, 0))],
        out_shape=[
            jax.ShapeDtypeStruct((e, DE), jnp.float32),
            jax.ShapeDtypeStruct((2, e, 128), jnp.float32),
        ],
    )(hs, bd, he, wa, wc, we2, wm1h, wm1e, wm2)


_NB = 2000


def _node_body(h_ref, agg_ref, wu1h, wu1a, wu2, wbn, h_out, btab_out):
    h = h_ref[...]
    f32 = jnp.float32
    u0 = jax.nn.relu(jnp.dot(h, wu1h[...], preferred_element_type=f32)
                     + jnp.dot(agg_ref[0], wu1a[...][:128],
                               preferred_element_type=f32)
                     + jnp.dot(agg_ref[1], wu1a[...][128:],
                               preferred_element_type=f32))
    u = jnp.dot(u0, wu2[...], preferred_element_type=f32)
    h2 = _ln(h + DELTA * u)
    h_out[...] = h2
    btab_out[...] = jnp.dot(h2, wbn[...], preferred_element_type=jnp.float32)


def _tc_node_update(h, agg, wu1h, wu1a, wu2, wbn):
    grid = (N // _NB,)
    full = lambda *s: pl.BlockSpec(s, lambda i: (0,) * len(s))
    row = lambda *s: pl.BlockSpec(s, lambda i: (i,) + (0,) * (len(s) - 1))
    return pl.pallas_call(
        _node_body,
        grid=grid,
        in_specs=[
            row(_NB, D),
            pl.BlockSpec((2, _NB, 128), lambda i: (0, i, 0)),
            full(D, MR * D), full(D, MR * D), full(MR * D, D),
            full(D, 128),
        ],
        out_specs=[row(_NB, D), row(_NB, 128)],
        out_shape=[
            jax.ShapeDtypeStruct((N, D), jnp.float32),
            jax.ShapeDtypeStruct((N, 128), jnp.float32),
        ],
    )(h, agg, wu1h, wu1a, wu2, wbn)


def _btab_body(h_ref, wb, out_ref):
    out_ref[...] = jnp.dot(h_ref[...], wb[...],
                           preferred_element_type=jnp.float32)


def _tc_btab(h, wb):
    return pl.pallas_call(
        _btab_body,
        grid=(N // _NB,),
        in_specs=[pl.BlockSpec((_NB, D), lambda i: (i, 0)),
                  pl.BlockSpec((D, 128), lambda i: (0, 0))],
        out_specs=pl.BlockSpec((_NB, 128), lambda i: (i, 0)),
        out_shape=jax.ShapeDtypeStruct((N, 128), jnp.float32),
    )(h, wb)


def _tf_body(bnn_ref, x_ref, pn_g, pn_b, wq, bq, wk, bk, wv, bv, wo, bo,
             wf1, bf1, wf2, bf2, l1g, l1b, l2g, l2b, wl, bl, lng, lnb,
             lf_ref, g_ref):
    bidx = pl.program_id(0)
    nb = bnn_ref[bidx]
    x0 = _ln(x_ref[0], pn_g[...], pn_b[...])
    q = _bdot(x0, wq[...]) + bq[...]
    k = _bdot(x0, wk[...]) + bk[...]
    v = _bdot(x0, wv[...]) + bv[...]
    colmask = lax.broadcasted_iota(jnp.int32, (MAXLEN, MAXLEN), 1) >= nb
    dh = D // H
    outs = []
    for hh in range(H):
        sl = slice(hh * dh, (hh + 1) * dh)
        s = lax.dot_general(q[:, sl].astype(jnp.bfloat16),
                            k[:, sl].astype(jnp.bfloat16),
                            (((1,), (1,)), ((), ())),
                            preferred_element_type=jnp.float32)
        s = s * np.float32(1.0 / np.sqrt(dh))
        s = jnp.where(colmask, np.float32(-1e9), s)
        s = s - jnp.max(s, axis=-1, keepdims=True)
        es = jnp.exp(s)
        a = es / jnp.sum(es, axis=-1, keepdims=True)
        outs.append(_bdot(a, v[:, sl]))
    o = jnp.concatenate(outs, axis=1)
    o = _bdot(o, wo[...]) + bo[...]
    x1 = _ln(x0 + o, l1g[...], l1b[...])
    f0 = jax.nn.relu(_bdot(x1, wf1[...]) + bf1[...])
    f = _bdot(f0, wf2[...]) + bf2[...]
    lf = _ln(x1 + f, l2g[...], l2b[...])
    lf_ref[0] = lf
    rowmask = lax.broadcasted_iota(jnp.int32, (MAXLEN, 1), 0) < nb
    pooled = (jnp.sum(jnp.where(rowmask, lf, 0.0), axis=0, keepdims=True)
              / nb.astype(jnp.float32))
    g = _ln(_bdot(pooled, wl[...]) + bl[...], lng[...], lnb[...])
    g_ref[0] = g


def _tc_transformer(h, bnn, tf, wl, bl, lng, lnb, pn_g, pn_b):
    xb = h.reshape(B, MAXLEN, D)
    r2 = lambda a: a.reshape(1, -1)
    full = lambda *s: pl.BlockSpec(s, lambda i: (0,) * len(s))
    args = [
        xb, r2(pn_g), r2(pn_b),
        tf["Wq"], r2(tf["bq"]), tf["Wk"], r2(tf["bk"]),
        tf["Wv"], r2(tf["bv"]), tf["Wo"], r2(tf["bo"]),
        tf["Wf1"], r2(tf["bf1"]), tf["Wf2"], r2(tf["bf2"]),
        r2(tf["ln1_g"]), r2(tf["ln1_b"]), r2(tf["ln2_g"]), r2(tf["ln2_b"]),
        wl, r2(bl), r2(lng), r2(lnb),
    ]
    in_specs = [pl.BlockSpec(memory_space=pltpu.SMEM),
                pl.BlockSpec((1, MAXLEN, D), lambda i: (i, 0, 0))]
    in_specs += [full(*a.shape) for a in args[1:]]
    lf, g = pl.pallas_call(
        _tf_body,
        grid=(B,),
        in_specs=in_specs,
        out_specs=[pl.BlockSpec((1, MAXLEN, D), lambda i: (i, 0, 0)),
                   pl.BlockSpec((1, 1, D), lambda i: (i, 0, 0))],
        out_shape=[
            jax.ShapeDtypeStruct((B, MAXLEN, D), jnp.float32),
            jax.ShapeDtypeStruct((B, 1, D), jnp.float32),
        ],
    )(bnn, *args)
    return lf.reshape(N, D), g.reshape(B, D)


_EA = 83200


def kernel(h, he, edge_index, batch_num_nodes, params):
    src = edge_index[0]
    dst = edge_index[1]
    halves = [
        (src[:_EA], dst[:_EA], he[:_EA]),
        (src[_EA:], dst[_EA:], he[_EA:]),
    ]
    zero_agg = jnp.zeros((2, N, 128), jnp.float32)

    pad_b = lambda w: jnp.pad(w, ((0, 0), (0, 128 - MR * DE)))
    wb_next = pad_b(params["edge1"]["We1"][D:2 * D])
    btab = _tc_btab(h, pad_b(params["edge0"]["We1"][D:2 * D]))

    for l in range(NUM_LAYERS):
        pe = params["edge0" if l == 0 else "edge1"]
        pn = params["node0" if l == 0 else "node1"]
        wa = pe["We1"][:D]
        wc = pe["We1"][2 * D:]
        wm1h = pn["Wm1"][:D]
        wm1e = pn["Wm1"][D:]
        wu1h = pn["Wu1"][:D]
        wu1a = pn["Wu1"][D:]

        msgs = []
        new_halves = []
        for (src_h, dst_h, he_h) in halves:
            hs = _sc_gather(h, src_h, chunk=200)
            bd = _sc_gather(btab, dst_h, chunk=200)
            he2, msg = _tc_edge_msg(hs, bd, he_h, wa, wc, we2=pe["We2"],
                                    wm1h=wm1h, wm1e=wm1e, wm2=pn["Wm2"])
            new_halves.append((src_h, dst_h, he2))
            msgs.append(msg)
        halves = new_halves
        agg = _sc_scatter_add(msgs[0], halves[0][1], zero_agg)
        agg = _sc_scatter_add(msgs[1], halves[1][1], agg)
        h, btab = _tc_node_update(h, agg, wu1h, wu1a, pn["Wu2"], wb_next)

    local_feat, global_feat = _tc_transformer(
        h, batch_num_nodes, params["tf"], params["Wl"], params["bl"],
        params["ln_g"], params["ln_b"], params["pn_g"], params["pn_b"])
    return local_feat, global_feat
